# baseline (speedup 1.0000x reference)
import functools

import jax
import jax.numpy as jnp
from jax.experimental import pallas as pl
from jax.experimental.pallas import tpu as pltpu


def _gcn_body(x_ref, adj_ref, w1_ref, b1_ref, w2_ref, b2_ref,
              feat_hbm, out_hbm,
              support_ref, hacc_ref, fstage, ostage, f_sems, o_sems,
              *, bn, bk, n_i):
    j = pl.program_id(0)
    i = pl.program_id(1)

    @pl.when(i == 0)
    def _compute_support_half():
        half = (
            jnp.dot(x_ref[...].astype(jnp.bfloat16),
                    w1_ref[...].astype(jnp.bfloat16),
                    preferred_element_type=jnp.float32)
            + b1_ref[...]
        ).astype(jnp.bfloat16)

        @pl.when(j == 0)
        def _lo():
            support_ref[:bk, :] = half

        @pl.when(j == 1)
        def _hi():
            support_ref[bk:, :] = half

    def feat_cp(blk, slot):
        return pltpu.make_async_copy(
            fstage.at[slot], feat_hbm.at[pl.ds(blk * bn, bn), :],
            f_sems.at[slot])

    def out_cp(blk, slot):
        return pltpu.make_async_copy(
            ostage.at[slot], out_hbm.at[pl.ds(blk * bn, bn), :],
            o_sems.at[slot])

    @pl.when(j == 0)
    def _first_pass():
        hacc_ref[pl.ds(i * bn, bn), :] = jnp.dot(
            adj_ref[...].astype(jnp.bfloat16), support_ref[:bk, :],
            preferred_element_type=jnp.float32)

    @pl.when(j == 1)
    def _finalize():
        h = hacc_ref[pl.ds(i * bn, bn), :] + jnp.dot(
            adj_ref[...].astype(jnp.bfloat16), support_ref[bk:, :],
            preferred_element_type=jnp.float32)
        feat = jnp.maximum(h, 0.0)

        @pl.when(i >= 2)
        def _reclaim():
            feat_cp(i - 2, i % 2).wait()
            out_cp(i - 2, i % 2).wait()

        fstage[i % 2] = feat
        ostage[i % 2] = jax.nn.sigmoid(
            jnp.dot(feat.astype(jnp.bfloat16), w2_ref[...].astype(jnp.bfloat16),
                    preferred_element_type=jnp.float32)
            + b2_ref[...]
        )
        feat_cp(i, i % 2).start()
        out_cp(i, i % 2).start()

        @pl.when(i == n_i - 1)
        def _drain():
            feat_cp(n_i - 2, (n_i - 2) % 2).wait()
            out_cp(n_i - 2, (n_i - 2) % 2).wait()
            feat_cp(n_i - 1, (n_i - 1) % 2).wait()
            out_cp(n_i - 1, (n_i - 1) % 2).wait()


@functools.partial(jax.jit, static_argnames=("block_n",))
def _gcn_fused(x, adj, W1, b1, W2, b2, block_n=1024):
    n, f = x.shape
    h_dim = W1.shape[1]
    c = W2.shape[1]
    bk = n // 2
    n_i = n // block_n
    b1r = b1.reshape(1, h_dim)
    b2r = b2.reshape(1, c)
    feature, out = pl.pallas_call(
        functools.partial(_gcn_body, bn=block_n, bk=bk, n_i=n_i),
        grid=(2, n_i),
        in_specs=[
            pl.BlockSpec((bk, f), lambda j, i: (j, 0)),
            pl.BlockSpec((block_n, bk), lambda j, i: (i, j)),
            pl.BlockSpec((f, h_dim), lambda j, i: (0, 0)),
            pl.BlockSpec((1, h_dim), lambda j, i: (0, 0)),
            pl.BlockSpec((h_dim, c), lambda j, i: (0, 0)),
            pl.BlockSpec((1, c), lambda j, i: (0, 0)),
        ],
        out_specs=[
            pl.BlockSpec(memory_space=pltpu.MemorySpace.HBM),
            pl.BlockSpec(memory_space=pltpu.MemorySpace.HBM),
        ],
        out_shape=[
            jax.ShapeDtypeStruct((n, h_dim), jnp.float32),
            jax.ShapeDtypeStruct((n, c), jnp.float32),
        ],
        scratch_shapes=[
            pltpu.VMEM((n, h_dim), jnp.bfloat16),
            pltpu.VMEM((n, h_dim), jnp.float32),
            pltpu.VMEM((2, block_n, h_dim), jnp.float32),
            pltpu.VMEM((2, block_n, c), jnp.float32),
            pltpu.SemaphoreType.DMA((2,)),
            pltpu.SemaphoreType.DMA((2,)),
        ],
        compiler_params=pltpu.CompilerParams(
            dimension_semantics=("arbitrary", "arbitrary"),
        ),
    )(x, adj, W1, b1r, W2, b2r)
    return feature, out


def kernel(x, adj, W1, b1, W2, b2):
    return _gcn_fused(x, adj, W1, b1, W2, b2)


# speedup vs baseline: 1.0388x; 1.0388x over previous
import functools

import jax
import jax.numpy as jnp
from jax.experimental import pallas as pl
from jax.experimental.pallas import tpu as pltpu


def _gcn_body(x_ref, adj_ref, w1_ref, b1_ref, w2_ref, b2_ref,
              feat_ref, out_ref, support_ref):
    i = pl.program_id(0)

    @pl.when(i == 0)
    def _compute_support():
        support_ref[...] = (
            jnp.dot(x_ref[...].astype(jnp.bfloat16),
                    w1_ref[...].astype(jnp.bfloat16),
                    preferred_element_type=jnp.float32)
            + b1_ref[...]
        ).astype(jnp.bfloat16)

    h = jnp.dot(adj_ref[...].astype(jnp.bfloat16), support_ref[...],
                preferred_element_type=jnp.float32)
    feat = jnp.maximum(h, 0.0)
    feat_ref[...] = feat
    out_ref[...] = jax.nn.sigmoid(
        jnp.dot(feat.astype(jnp.bfloat16), w2_ref[...].astype(jnp.bfloat16),
                preferred_element_type=jnp.float32)
        + b2_ref[...]
    )


@functools.partial(jax.jit, static_argnames=("block_n",))
def _gcn_fused(x, adj, W1, b1, W2, b2, block_n=512):
    n, f = x.shape
    h_dim = W1.shape[1]
    c = W2.shape[1]
    b1r = b1.reshape(1, h_dim)
    b2r = b2.reshape(1, c)
    feature, out = pl.pallas_call(
        _gcn_body,
        grid=(n // block_n,),
        in_specs=[
            pl.BlockSpec((n, f), lambda i: (0, 0)),
            pl.BlockSpec((block_n, n), lambda i: (i, 0)),
            pl.BlockSpec((f, h_dim), lambda i: (0, 0)),
            pl.BlockSpec((1, h_dim), lambda i: (0, 0)),
            pl.BlockSpec((h_dim, c), lambda i: (0, 0)),
            pl.BlockSpec((1, c), lambda i: (0, 0)),
        ],
        out_specs=[
            pl.BlockSpec((block_n, h_dim), lambda i: (i, 0)),
            pl.BlockSpec((block_n, c), lambda i: (i, 0)),
        ],
        out_shape=[
            jax.ShapeDtypeStruct((n, h_dim), jnp.float32),
            jax.ShapeDtypeStruct((n, c), jnp.float32),
        ],
        scratch_shapes=[pltpu.VMEM((n, h_dim), jnp.bfloat16)],
    )(x, adj, W1, b1r, W2, b2r)
    return feature, out


def kernel(x, adj, W1, b1, W2, b2):
    return _gcn_fused(x, adj, W1, b1, W2, b2, block_n=1024)
